# Initial kernel scaffold; baseline (speedup 1.0000x reference)
import jax
import jax.numpy as jnp
from jax.experimental import pallas as pl


def kernel(x, edge_index, batch, W1, b1, W2, b2, fW1, fb1, fW2, fb2):
    raise NotImplementedError("write your pallas kernel here")


# speedup vs baseline: 18.1120x; 18.1120x over previous
import jax
import jax.numpy as jnp
from jax import lax
from jax.experimental import pallas as pl
from jax.experimental.pallas import tpu as pltpu
from jax.experimental.pallas import tpu_sc as plsc

_NC = 2
_NS = 16
_NW = _NC * _NS
_C = 80


def _sc_mesh():
    return plsc.VectorSubcoreMesh(core_axis_name="c", subcore_axis_name="s")


def _round_up(v, m):
    return (v + m - 1) // m * m


def _hist(dst, n_nodes):
    e = dst.shape[0]
    epw = e // _NW
    assert epw * _NW == e and epw % _C == 0
    nchunks = epw // _C
    npad = _round_up(n_nodes, _NS * 16)
    rpt = npad // _NS

    def body(dst_hbm, out_hbm, idxv, onesv, zbuf, acc_sh):
        cid = lax.axis_index("c")
        sid = lax.axis_index("s")
        wid = sid * _NC + cid
        ones16 = jnp.ones((16,), jnp.float32)
        zeros16 = jnp.zeros((16,), jnp.float32)
        for i in range(_C // 16):
            onesv[pl.ds(i * 16, 16)] = ones16
        for i in range(rpt // 16):
            zbuf[pl.ds(i * 16, 16)] = zeros16
        pltpu.sync_copy(zbuf, acc_sh.at[pl.ds(sid * rpt, rpt)])
        plsc.subcore_barrier()
        base = wid * epw

        def chunk(j, carry):
            off = pl.multiple_of(base + j * _C, 8)
            pltpu.sync_copy(dst_hbm.at[pl.ds(off, _C)], idxv)
            pltpu.sync_copy(onesv, acc_sh.at[idxv], add=True)
            return carry

        lax.fori_loop(0, nchunks, chunk, 0)
        plsc.subcore_barrier()
        off = sid * rpt
        oout = pl.multiple_of(cid * npad + off, 8)
        pltpu.sync_copy(acc_sh.at[pl.ds(off, rpt)], zbuf)
        pltpu.sync_copy(zbuf, out_hbm.at[pl.ds(oout, rpt)])

    f = pl.kernel(
        body,
        out_type=jax.ShapeDtypeStruct((_NC * npad,), jnp.float32),
        mesh=_sc_mesh(),
        scratch_types=[
            pltpu.VMEM((_C,), jnp.int32),
            pltpu.VMEM((_C,), jnp.float32),
            pltpu.VMEM((rpt,), jnp.float32),
            pltpu.VMEM_SHARED((npad,), jnp.float32),
        ],
    )
    return f(dst)


def _edge_scatter(y, src, dst, shift, n_out):
    d = y.shape[1]
    e = src.shape[0]
    epw = e // _NW
    assert epw * _NW == e and epw % _C == 0
    nchunks = epw // _C
    npad = _round_up(n_out, _NS * 16)
    rpt = npad // _NS

    def body(y_hbm, src_hbm, dst_hbm, out_hbm,
             srcv, dstv, srcs, dsts, rows, zbuf, acc_sh, sem):
        cid = lax.axis_index("c")
        sid = lax.axis_index("s")
        wid = sid * _NC + cid
        zeros16 = jnp.zeros((16,), jnp.float32)
        for i in range(16):
            for j in range(d // 16):
                zbuf[i, pl.ds(j * 16, 16)] = zeros16
        for k in range(rpt // 16):
            pltpu.sync_copy(zbuf, acc_sh.at[pl.ds(sid * rpt + k * 16, 16)])
        plsc.subcore_barrier()
        base = wid * epw

        def chunk(j, carry):
            off = pl.multiple_of(base + j * _C, 8)
            pltpu.sync_copy(src_hbm.at[pl.ds(off, _C)], srcv)
            pltpu.sync_copy(dst_hbm.at[pl.ds(off, _C)], dstv)
            if shift:
                for i in range(_C // 16):
                    srcs[pl.ds(i * 16, 16)] = lax.shift_right_logical(
                        srcv[pl.ds(i * 16, 16)], shift)
                    dsts[pl.ds(i * 16, 16)] = lax.shift_right_logical(
                        dstv[pl.ds(i * 16, 16)], shift)
                gi, si = srcs, dsts
            else:
                gi, si = srcv, dstv
            pltpu.async_copy(y_hbm.at[gi], rows, sem).wait()
            pltpu.sync_copy(rows, acc_sh.at[si], add=True)
            return carry

        lax.fori_loop(0, nchunks, chunk, 0)
        plsc.subcore_barrier()
        off = sid * rpt
        oout = pl.multiple_of(cid * npad + off, 8)
        for k in range(rpt // _C):
            pltpu.sync_copy(acc_sh.at[pl.ds(off + k * _C, _C)], rows)
            pltpu.sync_copy(rows, out_hbm.at[pl.ds(oout + k * _C, _C)])

    f = pl.kernel(
        body,
        out_type=jax.ShapeDtypeStruct((_NC * npad, d), jnp.float32),
        mesh=_sc_mesh(),
        scratch_types=[
            pltpu.VMEM((_C,), jnp.int32),
            pltpu.VMEM((_C,), jnp.int32),
            pltpu.VMEM((_C,), jnp.int32),
            pltpu.VMEM((_C,), jnp.int32),
            pltpu.VMEM((_C, d), jnp.float32),
            pltpu.VMEM((16, d), jnp.float32),
            pltpu.VMEM_SHARED((npad, d), jnp.float32),
            pltpu.SemaphoreType.DMA,
        ],
    )
    return f(y, src, dst)


def _tc1_body(x_ref, w_ref, d1_ref, y_ref):
    h = jnp.dot(x_ref[...], w_ref[...], preferred_element_type=jnp.float32)
    y_ref[...] = h * d1_ref[...]


def _tc2_body(a0_ref, a1_ref, yp_ref, d1p_ref, b1_ref, w2_ref, d2_ref, out_ref):
    s = a0_ref[...] + a1_ref[...] + yp_ref[...]
    z = s * d1p_ref[...] + b1_ref[...]
    h = jnp.maximum(jnp.maximum(z[:, 0, :], z[:, 1, :]), 0.0)
    y2 = jnp.dot(h, w2_ref[...], preferred_element_type=jnp.float32)
    out_ref[...] = y2 * d2_ref[...]


def _tc3_body(a0_ref, a1_ref, yp_ref, d2p_ref, b2_ref, seg_ref,
              fw1_ref, fb1_ref, fw2_ref, fb2_ref, out_ref):
    s = a0_ref[...] + a1_ref[...] + yp_ref[...]
    z = s * d2p_ref[...] + b2_ref[...]
    h = jnp.maximum(jnp.maximum(z[:, 0, :], z[:, 1, :]), 0.0)
    oh = (lax.broadcasted_iota(jnp.int32, (8, h.shape[0]), 0)
          == seg_ref[...]).astype(jnp.float32)
    g = jnp.dot(oh, h, preferred_element_type=jnp.float32)
    g = jnp.maximum(
        jnp.dot(g, fw1_ref[...], preferred_element_type=jnp.float32)
        + fb1_ref[...], 0.0)
    out_ref[...] = jnp.maximum(
        jnp.dot(g, fw2_ref[...], preferred_element_type=jnp.float32)
        + fb2_ref[...], 0.0)


def kernel(x, edge_index, batch, W1, b1, W2, b2, fW1, fb1, fW2, fb2):
    n, d = x.shape
    src = edge_index[0]
    dst = edge_index[1]
    f32 = jnp.float32

    deg2sc = _hist(dst, n).reshape(_NC, -1)
    deg_raw = (deg2sc[0] + deg2sc[1])[:n]
    dinv1 = lax.rsqrt(deg_raw + 1.0)
    degp = deg_raw.reshape(n // 2, 2)
    dinv2 = lax.rsqrt(degp[:, 0] + degp[:, 1] + 1.0)

    y1 = pl.pallas_call(
        _tc1_body, out_shape=jax.ShapeDtypeStruct((n, d), f32),
    )(x, W1, dinv1[:, None])

    acc1 = _edge_scatter(y1, src, dst, shift=0, n_out=n).reshape(_NC, -1, d)
    a0 = acc1[0, :n].reshape(n // 2, 2, d)
    a1 = acc1[1, :n].reshape(n // 2, 2, d)

    y2 = pl.pallas_call(
        _tc2_body, out_shape=jax.ShapeDtypeStruct((n // 2, d), f32),
    )(a0, a1, y1.reshape(n // 2, 2, d), dinv1.reshape(n // 2, 2, 1),
      b1.reshape(1, 1, d), W2, dinv2[:, None])

    acc2 = _edge_scatter(y2, src, dst, shift=1, n_out=n // 2).reshape(_NC, -1, d)
    c0 = acc2[0, :n // 2].reshape(n // 4, 2, d)
    c1 = acc2[1, :n // 2].reshape(n // 4, 2, d)

    seg = batch[::4].reshape(1, n // 4)
    out = pl.pallas_call(
        _tc3_body, out_shape=jax.ShapeDtypeStruct((8, d), f32),
    )(c0, c1, y2.reshape(n // 4, 2, d), dinv2.reshape(n // 4, 2, 1),
      b2.reshape(1, 1, d), seg, fW1, fb1.reshape(1, d), fW2, fb2.reshape(1, d))
    return out


# speedup vs baseline: 35.7647x; 1.9746x over previous
import jax
import jax.numpy as jnp
from jax import lax
from jax.experimental import pallas as pl
from jax.experimental.pallas import tpu as pltpu
from jax.experimental.pallas import tpu_sc as plsc

_NC = 2
_NS = 16
_NW = _NC * _NS
_C = 80


def _sc_mesh():
    return plsc.VectorSubcoreMesh(core_axis_name="c", subcore_axis_name="s")


def _round_up(v, m):
    return (v + m - 1) // m * m


def _hist(dst, n_nodes):
    e = dst.shape[0]
    epw = e // _NW
    assert epw * _NW == e and epw % _C == 0
    nchunks = epw // _C
    npad = _round_up(n_nodes, _NS * 16)
    rpt = npad // _NS

    def body(dst_hbm, out_hbm, idxv, onesv, zbuf, acc_sh):
        cid = lax.axis_index("c")
        sid = lax.axis_index("s")
        wid = sid * _NC + cid
        ones16 = jnp.ones((16,), jnp.float32)
        zeros16 = jnp.zeros((16,), jnp.float32)
        for i in range(_C // 16):
            onesv[pl.ds(i * 16, 16)] = ones16
        for i in range(rpt // 16):
            zbuf[pl.ds(i * 16, 16)] = zeros16
        pltpu.sync_copy(zbuf, acc_sh.at[pl.ds(sid * rpt, rpt)])
        plsc.subcore_barrier()
        base = wid * epw

        def chunk(j, carry):
            off = pl.multiple_of(base + j * _C, 8)
            pltpu.sync_copy(dst_hbm.at[pl.ds(off, _C)], idxv)
            pltpu.sync_copy(onesv, acc_sh.at[idxv], add=True)
            return carry

        lax.fori_loop(0, nchunks, chunk, 0)
        plsc.subcore_barrier()
        off = sid * rpt
        oout = pl.multiple_of(cid * npad + off, 8)
        pltpu.sync_copy(acc_sh.at[pl.ds(off, rpt)], zbuf)
        pltpu.sync_copy(zbuf, out_hbm.at[pl.ds(oout, rpt)])

    f = pl.kernel(
        body,
        out_type=jax.ShapeDtypeStruct((_NC * npad,), jnp.float32),
        mesh=_sc_mesh(),
        scratch_types=[
            pltpu.VMEM((_C,), jnp.int32),
            pltpu.VMEM((_C,), jnp.float32),
            pltpu.VMEM((rpt,), jnp.float32),
            pltpu.VMEM_SHARED((npad,), jnp.float32),
        ],
    )
    return f(dst)


def _edge_scatter(y, src, dst, shift, n_out):
    d = y.shape[1]
    e = src.shape[0]
    epw = e // _NW
    assert epw * _NW == e and epw % _C == 0
    nchunks = epw // _C
    assert nchunks % 2 == 1 and nchunks >= 3
    npad = _round_up(n_out, _NS * 16)
    rpt = npad // _NS
    assert rpt % _C == 0

    def body(y_hbm, src_hbm, dst_hbm, out_hbm,
             srci, dsti, drow, rows0, rows1, zbuf, acc_sh,
             gsem0, gsem1, ssem0, ssem1):
        cid = lax.axis_index("c")
        sid = lax.axis_index("s")
        wid = sid * _NC + cid
        zeros16 = jnp.zeros((16,), jnp.float32)
        for i in range(16):
            for j in range(d // 16):
                zbuf[i, pl.ds(j * 16, 16)] = zeros16
        for k in range(rpt // 16):
            pltpu.sync_copy(zbuf, acc_sh.at[pl.ds(sid * rpt + k * 16, 16)])
        base = pl.multiple_of(wid * epw, 8)
        pltpu.sync_copy(src_hbm.at[pl.ds(base, epw)], srci)
        pltpu.sync_copy(dst_hbm.at[pl.ds(base, epw)], dsti)
        if shift:
            def sbody(k, c):
                sl = pl.ds(k * 16, 16)
                srci[sl] = lax.shift_right_logical(srci[sl], shift)
                dsti[sl] = lax.shift_right_logical(dsti[sl], shift)
                return c
            lax.fori_loop(0, epw // 16, sbody, 0)
        plsc.subcore_barrier()

        rows = (rows0, rows1)
        gsem = (gsem0, gsem1)
        ssem = (ssem0, ssem1)
        nbuf = len(rows)

        def gather(j, b):
            pltpu.async_copy(y_hbm.at[srci.at[pl.ds(j * _C, _C)]], rows[b],
                             gsem[b])

        def wait_g(j, b):
            pltpu.make_async_copy(y_hbm.at[srci.at[pl.ds(j * _C, _C)]],
                                  rows[b], gsem[b]).wait()

        def fill_d(j, b):
            for i in range(_C // 16):
                drow[b, pl.ds(i * 16, 16)] = dsti[pl.ds(j * _C + i * 16, 16)]

        def scatter(j, b):
            pltpu.async_copy(rows[b], acc_sh.at[drow.at[b]], ssem[b], add=True)

        def wait_s(b):
            pltpu.make_async_copy(rows[b], acc_sh.at[drow.at[b]],
                                  ssem[b]).wait()

        def full_step(j, b):
            wait_s(b)
            gather(j, b)
            bp = (b - 1) % nbuf
            wait_g(j - 1, bp)
            fill_d(j - 1, bp)
            scatter(j - 1, bp)

        for j in range(nbuf):
            gather(j, j)
            if j >= 1:
                bp = j - 1
                wait_g(j - 1, bp)
                fill_d(j - 1, bp)
                scatter(j - 1, bp)
        n_iter = (nchunks - nbuf) // nbuf
        rem = (nchunks - nbuf) % nbuf

        def pipe(k, c):
            j0 = nbuf + k * nbuf
            for i in range(nbuf):
                full_step(j0 + i, i)
            return c

        lax.fori_loop(0, n_iter, pipe, 0)
        for i in range(rem):
            full_step(nbuf + n_iter * nbuf + i, i)
        bp = (nchunks - 1) % nbuf
        wait_g(nchunks - 1, bp)
        fill_d(nchunks - 1, bp)
        scatter(nchunks - 1, bp)
        for b in range(nbuf):
            wait_s(b)
        plsc.subcore_barrier()
        off = sid * rpt
        oout = pl.multiple_of(cid * npad + off, 8)
        for k in range(rpt // _C):
            pltpu.sync_copy(acc_sh.at[pl.ds(off + k * _C, _C)], rows0)
            pltpu.sync_copy(rows0, out_hbm.at[pl.ds(oout + k * _C, _C)])

    f = pl.kernel(
        body,
        out_type=jax.ShapeDtypeStruct((_NC * npad, d), jnp.float32),
        mesh=_sc_mesh(),
        scratch_types=[
            pltpu.VMEM((epw,), jnp.int32),
            pltpu.VMEM((epw,), jnp.int32),
            pltpu.VMEM((2, _C), jnp.int32),
            pltpu.VMEM((_C, d), jnp.float32),
            pltpu.VMEM((_C, d), jnp.float32),
            pltpu.VMEM((16, d), jnp.float32),
            pltpu.VMEM_SHARED((npad, d), jnp.float32),
            pltpu.SemaphoreType.DMA,
            pltpu.SemaphoreType.DMA,
            pltpu.SemaphoreType.DMA,
            pltpu.SemaphoreType.DMA,
        ],
    )
    return f(y, src, dst)


def _tc1_body(x_ref, w_ref, d1_ref, y_ref):
    h = jnp.dot(x_ref[...], w_ref[...], preferred_element_type=jnp.float32)
    y_ref[...] = h * d1_ref[...]


def _tc2_body(a0_ref, a1_ref, yp_ref, d1p_ref, b1_ref, w2_ref, d2_ref, out_ref):
    s = a0_ref[...] + a1_ref[...] + yp_ref[...]
    z = s * d1p_ref[...] + b1_ref[...]
    h = jnp.maximum(jnp.maximum(z[:, 0, :], z[:, 1, :]), 0.0)
    y2 = jnp.dot(h, w2_ref[...], preferred_element_type=jnp.float32)
    out_ref[...] = y2 * d2_ref[...]


def _tc3_body(a0_ref, a1_ref, yp_ref, d2p_ref, b2_ref, seg_ref,
              fw1_ref, fb1_ref, fw2_ref, fb2_ref, out_ref):
    s = a0_ref[...] + a1_ref[...] + yp_ref[...]
    z = s * d2p_ref[...] + b2_ref[...]
    h = jnp.maximum(jnp.maximum(z[:, 0, :], z[:, 1, :]), 0.0)
    oh = (lax.broadcasted_iota(jnp.int32, (8, h.shape[0]), 0)
          == seg_ref[...]).astype(jnp.float32)
    g = jnp.dot(oh, h, preferred_element_type=jnp.float32)
    g = jnp.maximum(
        jnp.dot(g, fw1_ref[...], preferred_element_type=jnp.float32)
        + fb1_ref[...], 0.0)
    out_ref[...] = jnp.maximum(
        jnp.dot(g, fw2_ref[...], preferred_element_type=jnp.float32)
        + fb2_ref[...], 0.0)


def kernel(x, edge_index, batch, W1, b1, W2, b2, fW1, fb1, fW2, fb2):
    n, d = x.shape
    src = edge_index[0]
    dst = edge_index[1]
    f32 = jnp.float32

    deg2sc = _hist(dst, n).reshape(_NC, -1)
    deg_raw = (deg2sc[0] + deg2sc[1])[:n]
    dinv1 = lax.rsqrt(deg_raw + 1.0)
    degp = deg_raw.reshape(n // 2, 2)
    dinv2 = lax.rsqrt(degp[:, 0] + degp[:, 1] + 1.0)

    y1 = pl.pallas_call(
        _tc1_body, out_shape=jax.ShapeDtypeStruct((n, d), f32),
    )(x, W1, dinv1[:, None])

    acc1 = _edge_scatter(y1, src, dst, shift=0, n_out=n).reshape(_NC, -1, d)
    a0 = acc1[0, :n].reshape(n // 2, 2, d)
    a1 = acc1[1, :n].reshape(n // 2, 2, d)

    y2 = pl.pallas_call(
        _tc2_body, out_shape=jax.ShapeDtypeStruct((n // 2, d), f32),
    )(a0, a1, y1.reshape(n // 2, 2, d), dinv1.reshape(n // 2, 2, 1),
      b1.reshape(1, 1, d), W2, dinv2[:, None])

    acc2 = _edge_scatter(y2, src, dst, shift=1, n_out=n // 2).reshape(_NC, -1, d)
    c0 = acc2[0, :n // 2].reshape(n // 4, 2, d)
    c1 = acc2[1, :n // 2].reshape(n // 4, 2, d)

    seg = batch[::4].reshape(1, n // 4)
    out = pl.pallas_call(
        _tc3_body, out_shape=jax.ShapeDtypeStruct((8, d), f32),
    )(c0, c1, y2.reshape(n // 4, 2, d), dinv2.reshape(n // 4, 2, 1),
      b2.reshape(1, 1, d), seg, fW1, fb1.reshape(1, d), fW2, fb2.reshape(1, d))
    return out


# speedup vs baseline: 42.0618x; 1.1761x over previous
import jax
import jax.numpy as jnp
from jax import lax
from jax.experimental import pallas as pl
from jax.experimental.pallas import tpu as pltpu
from jax.experimental.pallas import tpu_sc as plsc

_NC = 2
_NS = 16
_NW = _NC * _NS
_C = 80


def _sc_mesh():
    return plsc.VectorSubcoreMesh(core_axis_name="c", subcore_axis_name="s")


def _round_up(v, m):
    return (v + m - 1) // m * m


def _hist(dst, n_nodes):
    e = dst.shape[0]
    epw = e // _NW
    assert epw * _NW == e and epw % _C == 0
    nchunks = epw // _C
    npad = _round_up(n_nodes, _NS * 16)
    rpt = npad // _NS

    assert nchunks * _C * 4 == epw * 4

    def body(dst_hbm, out_hbm, idxs, drows, onesv, zbuf, acc_sh, ssem):
        cid = lax.axis_index("c")
        sid = lax.axis_index("s")
        wid = sid * _NC + cid
        ones16 = jnp.ones((16,), jnp.float32)
        zeros16 = jnp.zeros((16,), jnp.float32)
        for i in range(_C // 16):
            onesv[pl.ds(i * 16, 16)] = ones16
        for i in range(rpt // 16):
            zbuf[pl.ds(i * 16, 16)] = zeros16
        base = pl.multiple_of(wid * epw, 8)
        pltpu.sync_copy(dst_hbm.at[pl.ds(base, epw)], idxs)
        pltpu.sync_copy(zbuf, acc_sh.at[pl.ds(sid * rpt, rpt)])

        def fill(k, c):
            for i in range(_C // 16):
                drows[k, pl.ds(i * 16, 16)] = idxs[pl.ds(k * _C + i * 16, 16)]
            return c

        lax.fori_loop(0, nchunks, fill, 0)
        plsc.subcore_barrier()

        def chunk(k, c):
            pltpu.async_copy(onesv, acc_sh.at[drows.at[k]], ssem, add=True)
            return c

        lax.fori_loop(0, nchunks, chunk, 0)
        pltpu.make_async_copy(dst_hbm.at[pl.ds(base, epw)], idxs, ssem).wait()
        plsc.subcore_barrier()
        off = sid * rpt
        oout = pl.multiple_of(cid * npad + off, 8)
        pltpu.sync_copy(acc_sh.at[pl.ds(off, rpt)], zbuf)
        pltpu.sync_copy(zbuf, out_hbm.at[pl.ds(oout, rpt)])

    f = pl.kernel(
        body,
        out_type=jax.ShapeDtypeStruct((_NC * npad,), jnp.float32),
        mesh=_sc_mesh(),
        scratch_types=[
            pltpu.VMEM((epw,), jnp.int32),
            pltpu.VMEM((nchunks, _C), jnp.int32),
            pltpu.VMEM((_C,), jnp.float32),
            pltpu.VMEM((rpt,), jnp.float32),
            pltpu.VMEM_SHARED((npad,), jnp.float32),
            pltpu.SemaphoreType.DMA,
        ],
    )
    return f(dst)


def _edge_scatter(y, src, dst, shift, n_out):
    d = y.shape[1]
    e = src.shape[0]
    epw = e // _NW
    assert epw * _NW == e and epw % _C == 0
    nchunks = epw // _C
    assert nchunks % 2 == 1 and nchunks >= 3
    npad = _round_up(n_out, _NS * 16)
    rpt = npad // _NS
    assert rpt % _C == 0

    def body(y_hbm, src_hbm, dst_hbm, out_hbm,
             srci, dsti, drow, rows0, rows1, rows2, zbuf, acc_sh,
             gsem0, gsem1, gsem2, ssem0, ssem1, ssem2):
        cid = lax.axis_index("c")
        sid = lax.axis_index("s")
        wid = sid * _NC + cid
        zeros16 = jnp.zeros((16,), jnp.float32)
        for i in range(16):
            for j in range(d // 16):
                zbuf[i, pl.ds(j * 16, 16)] = zeros16
        for k in range(rpt // 16):
            pltpu.sync_copy(zbuf, acc_sh.at[pl.ds(sid * rpt + k * 16, 16)])
        base = pl.multiple_of(wid * epw, 8)
        pltpu.sync_copy(src_hbm.at[pl.ds(base, epw)], srci)
        pltpu.sync_copy(dst_hbm.at[pl.ds(base, epw)], dsti)
        if shift:
            def sbody(k, c):
                sl = pl.ds(k * 16, 16)
                srci[sl] = lax.shift_right_logical(srci[sl], shift)
                dsti[sl] = lax.shift_right_logical(dsti[sl], shift)
                return c
            lax.fori_loop(0, epw // 16, sbody, 0)
        plsc.subcore_barrier()

        rows = (rows0, rows1)
        gsem = (gsem0, gsem1)
        ssem = (ssem0, ssem1)
        nbuf = len(rows)

        def gather(j, b):
            pltpu.async_copy(y_hbm.at[srci.at[pl.ds(j * _C, _C)]], rows[b],
                             gsem[b])

        def wait_g(j, b):
            pltpu.make_async_copy(y_hbm.at[srci.at[pl.ds(j * _C, _C)]],
                                  rows[b], gsem[b]).wait()

        def fill_d(j, b):
            for i in range(_C // 16):
                drow[b, pl.ds(i * 16, 16)] = dsti[pl.ds(j * _C + i * 16, 16)]

        def scatter(j, b):
            pltpu.async_copy(rows[b], acc_sh.at[drow.at[b]], ssem[b], add=True)

        def wait_s(b):
            pltpu.make_async_copy(rows[b], acc_sh.at[drow.at[b]],
                                  ssem[b]).wait()

        def full_step(j, b):
            wait_s(b)
            gather(j, b)
            bp = (b - 1) % nbuf
            wait_g(j - 1, bp)
            fill_d(j - 1, bp)
            scatter(j - 1, bp)

        for j in range(nbuf):
            gather(j, j)
            if j >= 1:
                bp = j - 1
                wait_g(j - 1, bp)
                fill_d(j - 1, bp)
                scatter(j - 1, bp)
        n_iter = (nchunks - nbuf) // nbuf
        rem = (nchunks - nbuf) % nbuf

        def pipe(k, c):
            j0 = nbuf + k * nbuf
            for i in range(nbuf):
                full_step(j0 + i, i)
            return c

        lax.fori_loop(0, n_iter, pipe, 0)
        for i in range(rem):
            full_step(nbuf + n_iter * nbuf + i, i)
        bp = (nchunks - 1) % nbuf
        wait_g(nchunks - 1, bp)
        fill_d(nchunks - 1, bp)
        scatter(nchunks - 1, bp)
        for b in range(nbuf):
            wait_s(b)
        plsc.subcore_barrier()
        off = sid * rpt
        oout = pl.multiple_of(cid * npad + off, 8)
        for k in range(rpt // _C):
            pltpu.sync_copy(acc_sh.at[pl.ds(off + k * _C, _C)], rows0)
            pltpu.sync_copy(rows0, out_hbm.at[pl.ds(oout + k * _C, _C)])

    f = pl.kernel(
        body,
        out_type=jax.ShapeDtypeStruct((_NC * npad, d), jnp.float32),
        mesh=_sc_mesh(),
        scratch_types=[
            pltpu.VMEM((epw,), jnp.int32),
            pltpu.VMEM((epw,), jnp.int32),
            pltpu.VMEM((3, _C), jnp.int32),
            pltpu.VMEM((_C, d), jnp.float32),
            pltpu.VMEM((_C, d), jnp.float32),
            pltpu.VMEM((_C, d), jnp.float32),
            pltpu.VMEM((16, d), jnp.float32),
            pltpu.VMEM_SHARED((npad, d), jnp.float32),
            pltpu.SemaphoreType.DMA,
            pltpu.SemaphoreType.DMA,
            pltpu.SemaphoreType.DMA,
            pltpu.SemaphoreType.DMA,
            pltpu.SemaphoreType.DMA,
            pltpu.SemaphoreType.DMA,
        ],
    )
    return f(y, src, dst)


def _tc1_body(x_ref, w_ref, d1_ref, y_ref):
    h = jnp.dot(x_ref[...], w_ref[...], preferred_element_type=jnp.float32)
    y_ref[...] = h * d1_ref[...]


def _tc2_body(a0_ref, a1_ref, yp_ref, d1p_ref, b1_ref, w2_ref, d2_ref, out_ref):
    s = a0_ref[...] + a1_ref[...] + yp_ref[...]
    z = s * d1p_ref[...] + b1_ref[...]
    h = jnp.maximum(jnp.maximum(z[:, 0, :], z[:, 1, :]), 0.0)
    y2 = jnp.dot(h, w2_ref[...], preferred_element_type=jnp.float32)
    out_ref[...] = y2 * d2_ref[...]


def _tc3_body(a0_ref, a1_ref, yp_ref, d2p_ref, b2_ref, seg_ref,
              fw1_ref, fb1_ref, fw2_ref, fb2_ref, out_ref):
    s = a0_ref[...] + a1_ref[...] + yp_ref[...]
    z = s * d2p_ref[...] + b2_ref[...]
    h = jnp.maximum(jnp.maximum(z[:, 0, :], z[:, 1, :]), 0.0)
    oh = (lax.broadcasted_iota(jnp.int32, (8, h.shape[0]), 0)
          == seg_ref[...]).astype(jnp.float32)
    g = jnp.dot(oh, h, preferred_element_type=jnp.float32)
    g = jnp.maximum(
        jnp.dot(g, fw1_ref[...], preferred_element_type=jnp.float32)
        + fb1_ref[...], 0.0)
    out_ref[...] = jnp.maximum(
        jnp.dot(g, fw2_ref[...], preferred_element_type=jnp.float32)
        + fb2_ref[...], 0.0)


def kernel(x, edge_index, batch, W1, b1, W2, b2, fW1, fb1, fW2, fb2):
    n, d = x.shape
    src = edge_index[0]
    dst = edge_index[1]
    f32 = jnp.float32

    deg2sc = _hist(dst, n).reshape(_NC, -1)
    deg_raw = (deg2sc[0] + deg2sc[1])[:n]
    dinv1 = lax.rsqrt(deg_raw + 1.0)
    degp = deg_raw.reshape(n // 2, 2)
    dinv2 = lax.rsqrt(degp[:, 0] + degp[:, 1] + 1.0)

    y1 = pl.pallas_call(
        _tc1_body, out_shape=jax.ShapeDtypeStruct((n, d), f32),
    )(x, W1, dinv1[:, None])

    acc1 = _edge_scatter(y1, src, dst, shift=0, n_out=n).reshape(_NC, -1, d)
    a0 = acc1[0, :n].reshape(n // 2, 2, d)
    a1 = acc1[1, :n].reshape(n // 2, 2, d)

    y2 = pl.pallas_call(
        _tc2_body, out_shape=jax.ShapeDtypeStruct((n // 2, d), f32),
    )(a0, a1, y1.reshape(n // 2, 2, d), dinv1.reshape(n // 2, 2, 1),
      b1.reshape(1, 1, d), W2, dinv2[:, None])

    acc2 = _edge_scatter(y2, src, dst, shift=1, n_out=n // 2).reshape(_NC, -1, d)
    c0 = acc2[0, :n // 2].reshape(n // 4, 2, d)
    c1 = acc2[1, :n // 2].reshape(n // 4, 2, d)

    seg = batch[::4].reshape(1, n // 4)
    out = pl.pallas_call(
        _tc3_body, out_shape=jax.ShapeDtypeStruct((8, d), f32),
    )(c0, c1, y2.reshape(n // 4, 2, d), dinv2.reshape(n // 4, 2, 1),
      b2.reshape(1, 1, d), seg, fW1, fb1.reshape(1, d), fW2, fb2.reshape(1, d))
    return out


# speedup vs baseline: 46.9864x; 1.1171x over previous
import jax
import jax.numpy as jnp
from jax import lax
from jax.experimental import pallas as pl
from jax.experimental.pallas import tpu as pltpu
from jax.experimental.pallas import tpu_sc as plsc

_NC = 2
_NS = 16
_NW = _NC * _NS
_C = 80


def _sc_mesh():
    return plsc.VectorSubcoreMesh(core_axis_name="c", subcore_axis_name="s")


def _round_up(v, m):
    return (v + m - 1) // m * m


def _hist(dst, n_nodes):
    e = dst.shape[0]
    epw = e // _NW
    assert epw * _NW == e and epw % _C == 0
    nchunks = epw // _C
    npad = _round_up(n_nodes, _NS * 16)
    rpt = npad // _NS

    assert nchunks * _C * 4 == epw * 4

    def body(dst_hbm, out_hbm, idxs, drows, onesv, zbuf, acc_sh, ssem):
        cid = lax.axis_index("c")
        sid = lax.axis_index("s")
        wid = sid * _NC + cid
        ones16 = jnp.ones((16,), jnp.float32)
        zeros16 = jnp.zeros((16,), jnp.float32)
        for i in range(_C // 16):
            onesv[pl.ds(i * 16, 16)] = ones16
        for i in range(rpt // 16):
            zbuf[pl.ds(i * 16, 16)] = zeros16
        base = pl.multiple_of(wid * epw, 8)
        pltpu.sync_copy(dst_hbm.at[pl.ds(base, epw)], idxs)
        pltpu.sync_copy(zbuf, acc_sh.at[pl.ds(sid * rpt, rpt)])

        def fill(k, c):
            for i in range(_C // 16):
                drows[k, pl.ds(i * 16, 16)] = idxs[pl.ds(k * _C + i * 16, 16)]
            return c

        lax.fori_loop(0, nchunks, fill, 0)
        plsc.subcore_barrier()

        def chunk(k, c):
            pltpu.async_copy(onesv, acc_sh.at[drows.at[k]], ssem, add=True)
            return c

        lax.fori_loop(0, nchunks, chunk, 0)
        pltpu.make_async_copy(dst_hbm.at[pl.ds(base, epw)], idxs, ssem).wait()
        plsc.subcore_barrier()
        off = sid * rpt
        oout = pl.multiple_of(cid * npad + off, 8)
        pltpu.sync_copy(acc_sh.at[pl.ds(off, rpt)], zbuf)
        pltpu.sync_copy(zbuf, out_hbm.at[pl.ds(oout, rpt)])

    f = pl.kernel(
        body,
        out_type=jax.ShapeDtypeStruct((_NC * npad,), jnp.float32),
        mesh=_sc_mesh(),
        scratch_types=[
            pltpu.VMEM((epw,), jnp.int32),
            pltpu.VMEM((nchunks, _C), jnp.int32),
            pltpu.VMEM((_C,), jnp.float32),
            pltpu.VMEM((rpt,), jnp.float32),
            pltpu.VMEM_SHARED((npad,), jnp.float32),
            pltpu.SemaphoreType.DMA,
        ],
    )
    return f(dst)


def _edge_scatter(y, src, dst, shift, n_out):
    d = y.shape[1]
    e = src.shape[0]
    dt = y.dtype
    lanes = 32 if dt == jnp.bfloat16 else 16
    epw = e // _NW
    assert epw * _NW == e and epw % _C == 0
    nchunks = epw // _C
    assert nchunks % 2 == 1 and nchunks >= 3
    npad = _round_up(n_out, _NS * 16)
    rpt = npad // _NS
    assert rpt % _C == 0

    def body(y_hbm, src_hbm, dst_hbm, out_hbm,
             srci, dsti, drow, rows0, rows1, rows2, zbuf, acc_sh,
             gsem0, gsem1, gsem2, ssem0, ssem1, ssem2):
        cid = lax.axis_index("c")
        sid = lax.axis_index("s")
        wid = sid * _NC + cid
        zerosv = jnp.zeros((lanes,), dt)
        for i in range(16):
            for j in range(d // lanes):
                zbuf[i, pl.ds(j * lanes, lanes)] = zerosv
        for k in range(rpt // 16):
            pltpu.sync_copy(zbuf, acc_sh.at[pl.ds(sid * rpt + k * 16, 16)])
        base = pl.multiple_of(wid * epw, 8)
        pltpu.sync_copy(src_hbm.at[pl.ds(base, epw)], srci)
        pltpu.sync_copy(dst_hbm.at[pl.ds(base, epw)], dsti)
        if shift:
            def sbody(k, c):
                sl = pl.ds(k * 16, 16)
                srci[sl] = lax.shift_right_logical(srci[sl], shift)
                dsti[sl] = lax.shift_right_logical(dsti[sl], shift)
                return c
            lax.fori_loop(0, epw // 16, sbody, 0)
        plsc.subcore_barrier()

        rows = (rows0, rows1)
        gsem = (gsem0, gsem1)
        ssem = (ssem0, ssem1)
        nbuf = len(rows)

        def gather(j, b):
            pltpu.async_copy(y_hbm.at[srci.at[pl.ds(j * _C, _C)]], rows[b],
                             gsem[b])

        def wait_g(j, b):
            pltpu.make_async_copy(y_hbm.at[srci.at[pl.ds(j * _C, _C)]],
                                  rows[b], gsem[b]).wait()

        def fill_d(j, b):
            for i in range(_C // 16):
                drow[b, pl.ds(i * 16, 16)] = dsti[pl.ds(j * _C + i * 16, 16)]

        def scatter(j, b):
            pltpu.async_copy(rows[b], acc_sh.at[drow.at[b]], ssem[b], add=True)

        def wait_s(b):
            pltpu.make_async_copy(rows[b], acc_sh.at[drow.at[b]],
                                  ssem[b]).wait()

        def full_step(j, b):
            wait_s(b)
            gather(j, b)
            bp = (b - 1) % nbuf
            wait_g(j - 1, bp)
            fill_d(j - 1, bp)
            scatter(j - 1, bp)

        for j in range(nbuf):
            gather(j, j)
            if j >= 1:
                bp = j - 1
                wait_g(j - 1, bp)
                fill_d(j - 1, bp)
                scatter(j - 1, bp)
        n_iter = (nchunks - nbuf) // nbuf
        rem = (nchunks - nbuf) % nbuf

        def pipe(k, c):
            j0 = nbuf + k * nbuf
            for i in range(nbuf):
                full_step(j0 + i, i)
            return c

        lax.fori_loop(0, n_iter, pipe, 0)
        for i in range(rem):
            full_step(nbuf + n_iter * nbuf + i, i)
        bp = (nchunks - 1) % nbuf
        wait_g(nchunks - 1, bp)
        fill_d(nchunks - 1, bp)
        scatter(nchunks - 1, bp)
        for b in range(nbuf):
            wait_s(b)
        plsc.subcore_barrier()
        off = sid * rpt
        oout = pl.multiple_of(cid * npad + off, 8)
        for k in range(rpt // _C):
            pltpu.sync_copy(acc_sh.at[pl.ds(off + k * _C, _C)], rows0)
            pltpu.sync_copy(rows0, out_hbm.at[pl.ds(oout + k * _C, _C)])

    f = pl.kernel(
        body,
        out_type=jax.ShapeDtypeStruct((_NC * npad, d), dt),
        mesh=_sc_mesh(),
        compiler_params=pltpu.CompilerParams(use_tc_tiling_on_sc=False),
        scratch_types=[
            pltpu.VMEM((epw,), jnp.int32),
            pltpu.VMEM((epw,), jnp.int32),
            pltpu.VMEM((3, _C), jnp.int32),
            pltpu.VMEM((_C, d), dt),
            pltpu.VMEM((_C, d), dt),
            pltpu.VMEM((_C, d), dt),
            pltpu.VMEM((16, d), dt),
            pltpu.VMEM_SHARED((npad, d), dt),
            pltpu.SemaphoreType.DMA,
            pltpu.SemaphoreType.DMA,
            pltpu.SemaphoreType.DMA,
            pltpu.SemaphoreType.DMA,
            pltpu.SemaphoreType.DMA,
            pltpu.SemaphoreType.DMA,
        ],
    )
    return f(y, src, dst)


def _tc1_body(x_ref, w_ref, d1_ref, y_ref):
    h = jnp.dot(x_ref[...], w_ref[...], preferred_element_type=jnp.float32)
    y_ref[...] = (h * d1_ref[...]).astype(y_ref.dtype)


def _tc2_body(a0_ref, a1_ref, yp_ref, d1p_ref, b1_ref, w2_ref, d2_ref, out_ref):
    f32 = jnp.float32
    s = (a0_ref[...].astype(f32) + a1_ref[...].astype(f32)
         + yp_ref[...].astype(f32))
    z = s * d1p_ref[...] + b1_ref[...]
    h = jnp.maximum(jnp.maximum(z[:, 0, :], z[:, 1, :]), 0.0)
    y2 = jnp.dot(h, w2_ref[...], preferred_element_type=jnp.float32)
    out_ref[...] = (y2 * d2_ref[...]).astype(out_ref.dtype)


def _tc3_body(a0_ref, a1_ref, yp_ref, d2p_ref, b2_ref, seg_ref,
              fw1_ref, fb1_ref, fw2_ref, fb2_ref, out_ref):
    f32 = jnp.float32
    s = (a0_ref[...].astype(f32) + a1_ref[...].astype(f32)
         + yp_ref[...].astype(f32))
    z = s * d2p_ref[...] + b2_ref[...]
    h = jnp.maximum(jnp.maximum(z[:, 0, :], z[:, 1, :]), 0.0)
    oh = (lax.broadcasted_iota(jnp.int32, (8, h.shape[0]), 0)
          == seg_ref[...]).astype(jnp.float32)
    g = jnp.dot(oh, h, preferred_element_type=jnp.float32)
    g = jnp.maximum(
        jnp.dot(g, fw1_ref[...], preferred_element_type=jnp.float32)
        + fb1_ref[...], 0.0)
    out_ref[...] = jnp.maximum(
        jnp.dot(g, fw2_ref[...], preferred_element_type=jnp.float32)
        + fb2_ref[...], 0.0)


def kernel(x, edge_index, batch, W1, b1, W2, b2, fW1, fb1, fW2, fb2):
    n, d = x.shape
    src = edge_index[0]
    dst = edge_index[1]
    f32 = jnp.float32

    deg2sc = _hist(dst, n).reshape(_NC, -1)
    deg_raw = (deg2sc[0] + deg2sc[1])[:n]
    dinv1 = lax.rsqrt(deg_raw + 1.0)
    degp = deg_raw.reshape(n // 2, 2)
    dinv2 = lax.rsqrt(degp[:, 0] + degp[:, 1] + 1.0)

    bf16 = jnp.bfloat16
    y1 = pl.pallas_call(
        _tc1_body, out_shape=jax.ShapeDtypeStruct((n, d), bf16),
    )(x, W1, dinv1[:, None])

    acc1 = _edge_scatter(y1, src, dst, shift=0, n_out=n).reshape(_NC, -1, d)
    a0 = acc1[0, :n].reshape(n // 2, 2, d)
    a1 = acc1[1, :n].reshape(n // 2, 2, d)

    y2 = pl.pallas_call(
        _tc2_body, out_shape=jax.ShapeDtypeStruct((n // 2, d), bf16),
    )(a0, a1, y1.reshape(n // 2, 2, d), dinv1.reshape(n // 2, 2, 1),
      b1.reshape(1, 1, d), W2, dinv2[:, None])

    acc2 = _edge_scatter(y2, src, dst, shift=1, n_out=n // 2).reshape(_NC, -1, d)
    c0 = acc2[0, :n // 2].reshape(n // 4, 2, d)
    c1 = acc2[1, :n // 2].reshape(n // 4, 2, d)

    seg = batch[::4].reshape(1, n // 4)
    out = pl.pallas_call(
        _tc3_body, out_shape=jax.ShapeDtypeStruct((8, d), f32),
    )(c0, c1, y2.reshape(n // 4, 2, d), dinv2.reshape(n // 4, 2, 1),
      b2.reshape(1, 1, d), seg, fW1, fb1.reshape(1, d), fW2, fb2.reshape(1, d))
    return out


# speedup vs baseline: 51.4159x; 1.0943x over previous
import jax
import jax.numpy as jnp
from jax import lax
from jax.experimental import pallas as pl
from jax.experimental.pallas import tpu as pltpu
from jax.experimental.pallas import tpu_sc as plsc

_NC = 2
_NS = 16
_NW = _NC * _NS
_C = 80


def _sc_mesh():
    return plsc.VectorSubcoreMesh(core_axis_name="c", subcore_axis_name="s")


def _round_up(v, m):
    return (v + m - 1) // m * m


def _hist(eidx, n_nodes):
    e = eidx.shape[1]
    epw = e // _NW
    assert epw * _NW == e and epw % _C == 0
    nchunks = epw // _C
    npad = _round_up(n_nodes, _NS * 16)
    rpt = npad // _NS

    assert nchunks * _C * 4 == epw * 4

    def body(ei_hbm, out_hbm, idxs, drows, onesv, zbuf, acc_sh, ssem):
        cid = lax.axis_index("c")
        sid = lax.axis_index("s")
        wid = sid * _NC + cid
        ones16 = jnp.ones((16,), jnp.float32)
        zeros16 = jnp.zeros((16,), jnp.float32)
        for i in range(_C // 16):
            onesv[pl.ds(i * 16, 16)] = ones16
        for i in range(rpt // 16):
            zbuf[pl.ds(i * 16, 16)] = zeros16
        base = pl.multiple_of(wid * epw, 8)
        pltpu.sync_copy(ei_hbm.at[1, pl.ds(base, epw)], idxs)
        pltpu.sync_copy(zbuf, acc_sh.at[pl.ds(sid * rpt, rpt)])

        def fill(k, c):
            for i in range(_C // 16):
                drows[k, pl.ds(i * 16, 16)] = idxs[pl.ds(k * _C + i * 16, 16)]
            return c

        lax.fori_loop(0, nchunks, fill, 0)
        plsc.subcore_barrier()

        def chunk(k, c):
            pltpu.async_copy(onesv, acc_sh.at[drows.at[k]], ssem, add=True)
            return c

        lax.fori_loop(0, nchunks, chunk, 0)
        pltpu.make_async_copy(ei_hbm.at[1, pl.ds(base, epw)], idxs, ssem).wait()
        plsc.subcore_barrier()
        off = sid * rpt
        oout = pl.multiple_of(cid * npad + off, 8)
        pltpu.sync_copy(acc_sh.at[pl.ds(off, rpt)], zbuf)
        pltpu.sync_copy(zbuf, out_hbm.at[pl.ds(oout, rpt)])

    f = pl.kernel(
        body,
        out_type=jax.ShapeDtypeStruct((_NC * npad,), jnp.float32),
        mesh=_sc_mesh(),
        compiler_params=pltpu.CompilerParams(use_tc_tiling_on_sc=False),
        scratch_types=[
            pltpu.VMEM((epw,), jnp.int32),
            pltpu.VMEM((nchunks, _C), jnp.int32),
            pltpu.VMEM((_C,), jnp.float32),
            pltpu.VMEM((rpt,), jnp.float32),
            pltpu.VMEM_SHARED((npad,), jnp.float32),
            pltpu.SemaphoreType.DMA,
        ],
    )
    return f(eidx)


def _edge_scatter(y, eidx, shift, n_out):
    d = y.shape[1]
    e = eidx.shape[1]
    dt = y.dtype
    lanes = 32 if dt == jnp.bfloat16 else 16
    epw = e // _NW
    assert epw * _NW == e and epw % _C == 0
    nchunks = epw // _C
    assert nchunks % 2 == 1 and nchunks >= 3
    npad = _round_up(n_out, _NS * 16)
    rpt = npad // _NS
    assert rpt % _C == 0

    def body(y_hbm, ei_hbm, out_hbm,
             srci, dsti, drow, rows0, rows1, rows2, zbuf, acc_sh,
             gsem0, gsem1, gsem2, ssem0, ssem1, ssem2):
        cid = lax.axis_index("c")
        sid = lax.axis_index("s")
        wid = sid * _NC + cid
        zerosv = jnp.zeros((lanes,), dt)
        for i in range(16):
            for j in range(d // lanes):
                zbuf[i, pl.ds(j * lanes, lanes)] = zerosv
        for k in range(rpt // 16):
            pltpu.sync_copy(zbuf, acc_sh.at[pl.ds(sid * rpt + k * 16, 16)])
        base = pl.multiple_of(wid * epw, 8)
        pltpu.sync_copy(ei_hbm.at[0, pl.ds(base, epw)], srci)
        pltpu.sync_copy(ei_hbm.at[1, pl.ds(base, epw)], dsti)
        if shift:
            def sbody(k, c):
                sl = pl.ds(k * 16, 16)
                srci[sl] = lax.shift_right_logical(srci[sl], shift)
                dsti[sl] = lax.shift_right_logical(dsti[sl], shift)
                return c
            lax.fori_loop(0, epw // 16, sbody, 0)
        plsc.subcore_barrier()

        rows = (rows0, rows1)
        gsem = (gsem0, gsem1)
        ssem = (ssem0, ssem1)
        nbuf = len(rows)

        def gather(j, b):
            pltpu.async_copy(y_hbm.at[srci.at[pl.ds(j * _C, _C)]], rows[b],
                             gsem[b])

        def wait_g(j, b):
            pltpu.make_async_copy(y_hbm.at[srci.at[pl.ds(j * _C, _C)]],
                                  rows[b], gsem[b]).wait()

        def fill_d(j, b):
            for i in range(_C // 16):
                drow[b, pl.ds(i * 16, 16)] = dsti[pl.ds(j * _C + i * 16, 16)]

        def scatter(j, b):
            pltpu.async_copy(rows[b], acc_sh.at[drow.at[b]], ssem[b], add=True)

        def wait_s(b):
            pltpu.make_async_copy(rows[b], acc_sh.at[drow.at[b]],
                                  ssem[b]).wait()

        def full_step(j, b):
            wait_s(b)
            gather(j, b)
            bp = (b - 1) % nbuf
            wait_g(j - 1, bp)
            fill_d(j - 1, bp)
            scatter(j - 1, bp)

        for j in range(nbuf):
            gather(j, j)
            if j >= 1:
                bp = j - 1
                wait_g(j - 1, bp)
                fill_d(j - 1, bp)
                scatter(j - 1, bp)
        n_iter = (nchunks - nbuf) // nbuf
        rem = (nchunks - nbuf) % nbuf

        def pipe(k, c):
            j0 = nbuf + k * nbuf
            for i in range(nbuf):
                full_step(j0 + i, i)
            return c

        lax.fori_loop(0, n_iter, pipe, 0)
        for i in range(rem):
            full_step(nbuf + n_iter * nbuf + i, i)
        bp = (nchunks - 1) % nbuf
        wait_g(nchunks - 1, bp)
        fill_d(nchunks - 1, bp)
        scatter(nchunks - 1, bp)
        for b in range(nbuf):
            wait_s(b)
        plsc.subcore_barrier()
        off = sid * rpt
        oout = pl.multiple_of(cid * npad + off, 8)
        for k in range(rpt // _C):
            pltpu.sync_copy(acc_sh.at[pl.ds(off + k * _C, _C)], rows0)
            pltpu.sync_copy(rows0, out_hbm.at[pl.ds(oout + k * _C, _C)])

    f = pl.kernel(
        body,
        out_type=jax.ShapeDtypeStruct((_NC * npad, d), dt),
        mesh=_sc_mesh(),
        compiler_params=pltpu.CompilerParams(use_tc_tiling_on_sc=False),
        scratch_types=[
            pltpu.VMEM((epw,), jnp.int32),
            pltpu.VMEM((epw,), jnp.int32),
            pltpu.VMEM((3, _C), jnp.int32),
            pltpu.VMEM((_C, d), dt),
            pltpu.VMEM((_C, d), dt),
            pltpu.VMEM((_C, d), dt),
            pltpu.VMEM((16, d), dt),
            pltpu.VMEM_SHARED((npad, d), dt),
            pltpu.SemaphoreType.DMA,
            pltpu.SemaphoreType.DMA,
            pltpu.SemaphoreType.DMA,
            pltpu.SemaphoreType.DMA,
            pltpu.SemaphoreType.DMA,
            pltpu.SemaphoreType.DMA,
        ],
    )
    return f(y, eidx)


def _tc1_body(x_ref, w_ref, d1_ref, y_ref):
    h = jnp.dot(x_ref[...], w_ref[...], preferred_element_type=jnp.float32)
    y_ref[...] = (h * d1_ref[...]).astype(y_ref.dtype)


def _combine_coarsen(acc_ref, y_ref, d_ref, b_ref):
    f32 = jnp.float32
    n, d = y_ref.shape
    npad = acc_ref.shape[0] // _NC
    s = (acc_ref[pl.ds(0, n), :].astype(f32)
         + acc_ref[pl.ds(npad, n), :].astype(f32)
         + y_ref[...].astype(f32))
    z = s * d_ref[...] + b_ref[...]
    z2 = z.reshape(n // 2, 2 * d)
    return jnp.maximum(jnp.maximum(z2[:, :d], z2[:, d:]), 0.0)


def _tc2_body(acc_ref, y1_ref, d1_ref, b1_ref, w2_ref, d2_ref, out_ref):
    h = _combine_coarsen(acc_ref, y1_ref, d1_ref, b1_ref)
    y2 = jnp.dot(h, w2_ref[...], preferred_element_type=jnp.float32)
    out_ref[...] = (y2 * d2_ref[...]).astype(out_ref.dtype)


def _tc3_body(acc_ref, y2_ref, d2_ref, b2_ref, seg_ref,
              fw1_ref, fb1_ref, fw2_ref, fb2_ref, out_ref):
    h = _combine_coarsen(acc_ref, y2_ref, d2_ref, b2_ref)
    oh = (lax.broadcasted_iota(jnp.int32, (8, h.shape[0]), 0)
          == seg_ref[...]).astype(jnp.float32)
    g = jnp.dot(oh, h, preferred_element_type=jnp.float32)
    g = jnp.maximum(
        jnp.dot(g, fw1_ref[...], preferred_element_type=jnp.float32)
        + fb1_ref[...], 0.0)
    out_ref[...] = jnp.maximum(
        jnp.dot(g, fw2_ref[...], preferred_element_type=jnp.float32)
        + fb2_ref[...], 0.0)


def kernel(x, edge_index, batch, W1, b1, W2, b2, fW1, fb1, fW2, fb2):
    n, d = x.shape
    f32 = jnp.float32
    bf16 = jnp.bfloat16

    deg2sc = _hist(edge_index, n)
    npadh = deg2sc.shape[0] // _NC
    deg_raw = (deg2sc[:npadh] + deg2sc[npadh:])[:n]
    dinv1 = lax.rsqrt(deg_raw + 1.0)
    degp = deg_raw.reshape(n // 2, 2)
    dinv2 = lax.rsqrt(degp[:, 0] + degp[:, 1] + 1.0)

    y1 = pl.pallas_call(
        _tc1_body, out_shape=jax.ShapeDtypeStruct((n, d), bf16),
    )(x, W1, dinv1[:, None])

    acc1 = _edge_scatter(y1, edge_index, shift=0, n_out=n)

    y2 = pl.pallas_call(
        _tc2_body, out_shape=jax.ShapeDtypeStruct((n // 2, d), bf16),
    )(acc1, y1, dinv1[:, None], b1.reshape(1, d), W2, dinv2[:, None])

    acc2 = _edge_scatter(y2, edge_index, shift=1, n_out=n // 2)

    seg = batch[::4].reshape(1, n // 4)
    out = pl.pallas_call(
        _tc3_body, out_shape=jax.ShapeDtypeStruct((8, d), f32),
    )(acc2, y2, dinv2[:, None], b2.reshape(1, d), seg,
      fW1, fb1.reshape(1, d), fW2, fb2.reshape(1, d))
    return out


# speedup vs baseline: 56.9973x; 1.1086x over previous
import jax
import jax.numpy as jnp
from jax import lax
from jax.experimental import pallas as pl
from jax.experimental.pallas import tpu as pltpu
from jax.experimental.pallas import tpu_sc as plsc

_NC = 2
_NS = 16
_NW = _NC * _NS
_C = 80


def _sc_mesh():
    return plsc.VectorSubcoreMesh(core_axis_name="c", subcore_axis_name="s")


def _round_up(v, m):
    return (v + m - 1) // m * m


def _hist(eidx, n_nodes):
    e = eidx.shape[1]
    epw = e // _NW
    assert epw * _NW == e and epw % _C == 0
    nchunks = epw // _C
    npad = _round_up(n_nodes, _NS * 16)
    rpt = npad // _NS

    assert nchunks * _C * 4 == epw * 4

    def body(ei_hbm, out_hbm, idxs, drows, onesv, zbuf, acc_sh, ssem):
        cid = lax.axis_index("c")
        sid = lax.axis_index("s")
        wid = sid * _NC + cid
        ones16 = jnp.ones((16,), jnp.float32)
        zeros16 = jnp.zeros((16,), jnp.float32)
        for i in range(_C // 16):
            onesv[pl.ds(i * 16, 16)] = ones16
        for i in range(rpt // 16):
            zbuf[pl.ds(i * 16, 16)] = zeros16
        base = pl.multiple_of(wid * epw, 8)
        pltpu.sync_copy(ei_hbm.at[1, pl.ds(base, epw)], idxs)
        pltpu.sync_copy(zbuf, acc_sh.at[pl.ds(sid * rpt, rpt)])

        def fill(k, c):
            for i in range(_C // 16):
                drows[k, pl.ds(i * 16, 16)] = idxs[pl.ds(k * _C + i * 16, 16)]
            return c

        lax.fori_loop(0, nchunks, fill, 0)
        plsc.subcore_barrier()

        def chunk(k, c):
            pltpu.async_copy(onesv, acc_sh.at[drows.at[k]], ssem, add=True)
            return c

        lax.fori_loop(0, nchunks, chunk, 0)
        pltpu.make_async_copy(ei_hbm.at[1, pl.ds(base, epw)], idxs, ssem).wait()
        plsc.subcore_barrier()
        off = sid * rpt
        oout = pl.multiple_of(cid * npad + off, 8)
        pltpu.sync_copy(acc_sh.at[pl.ds(off, rpt)], zbuf)
        pltpu.sync_copy(zbuf, out_hbm.at[pl.ds(oout, rpt)])

    f = pl.kernel(
        body,
        out_type=jax.ShapeDtypeStruct((_NC * npad,), jnp.float32),
        mesh=_sc_mesh(),
        compiler_params=pltpu.CompilerParams(use_tc_tiling_on_sc=False),
        scratch_types=[
            pltpu.VMEM((epw,), jnp.int32),
            pltpu.VMEM((nchunks, _C), jnp.int32),
            pltpu.VMEM((_C,), jnp.float32),
            pltpu.VMEM((rpt,), jnp.float32),
            pltpu.VMEM_SHARED((npad,), jnp.float32),
            pltpu.SemaphoreType.DMA,
        ],
    )
    return f(eidx)


def _edge_scatter(y, eidx, shift, n_out):
    d = y.shape[1]
    e = eidx.shape[1]
    dt = y.dtype
    lanes = 32 if dt == jnp.bfloat16 else 16
    epw = e // _NW
    assert epw * _NW == e and epw % _C == 0
    nchunks = epw // _C
    assert nchunks % 2 == 1 and nchunks >= 3
    npad = _round_up(n_out, _NS * 16)
    rpt = npad // _NS
    assert rpt % _C == 0

    def body(y_hbm, ei_hbm, out_hbm,
             srci, dsti, drow, rows0, rows1, rows2, zbuf, acc_sh,
             gsem0, gsem1, gsem2, ssem0, ssem1, ssem2):
        cid = lax.axis_index("c")
        sid = lax.axis_index("s")
        wid = sid * _NC + cid
        zerosv = jnp.zeros((lanes,), dt)
        for i in range(16):
            for j in range(d // lanes):
                zbuf[i, pl.ds(j * lanes, lanes)] = zerosv
        for k in range(rpt // 16):
            pltpu.sync_copy(zbuf, acc_sh.at[pl.ds(sid * rpt + k * 16, 16)])
        base = pl.multiple_of(wid * epw, 8)
        pltpu.sync_copy(ei_hbm.at[0, pl.ds(base, epw)], srci)
        pltpu.sync_copy(ei_hbm.at[1, pl.ds(base, epw)], dsti)
        if shift:
            def sbody(k, c):
                sl = pl.ds(k * 16, 16)
                srci[sl] = lax.shift_right_logical(srci[sl], shift)
                dsti[sl] = lax.shift_right_logical(dsti[sl], shift)
                return c
            lax.fori_loop(0, epw // 16, sbody, 0)
        plsc.subcore_barrier()

        rows = (rows0, rows1, rows2)
        gsem = (gsem0, gsem1, gsem2)
        ssem = (ssem0, ssem1, ssem2)
        nbuf = len(rows)

        def gather(j, b):
            pltpu.async_copy(y_hbm.at[srci.at[pl.ds(j * _C, _C)]], rows[b],
                             gsem[b])

        def wait_g(j, b):
            pltpu.make_async_copy(y_hbm.at[srci.at[pl.ds(j * _C, _C)]],
                                  rows[b], gsem[b]).wait()

        def fill_d(j, b):
            for i in range(_C // 16):
                drow[b, pl.ds(i * 16, 16)] = dsti[pl.ds(j * _C + i * 16, 16)]

        def scatter(j, b):
            pltpu.async_copy(rows[b], acc_sh.at[drow.at[b]], ssem[b], add=True)

        def wait_s(b):
            pltpu.make_async_copy(rows[b], acc_sh.at[drow.at[b]],
                                  ssem[b]).wait()

        def full_step(j, b):
            wait_s(b)
            gather(j, b)
            bp = (b - 1) % nbuf
            wait_g(j - 1, bp)
            fill_d(j - 1, bp)
            scatter(j - 1, bp)

        for j in range(nbuf):
            gather(j, j)
            if j >= 1:
                bp = j - 1
                wait_g(j - 1, bp)
                fill_d(j - 1, bp)
                scatter(j - 1, bp)
        n_iter = (nchunks - nbuf) // nbuf
        rem = (nchunks - nbuf) % nbuf

        def pipe(k, c):
            j0 = nbuf + k * nbuf
            for i in range(nbuf):
                full_step(j0 + i, i)
            return c

        lax.fori_loop(0, n_iter, pipe, 0)
        for i in range(rem):
            full_step(nbuf + n_iter * nbuf + i, i)
        bp = (nchunks - 1) % nbuf
        wait_g(nchunks - 1, bp)
        fill_d(nchunks - 1, bp)
        scatter(nchunks - 1, bp)
        for b in range(nbuf):
            wait_s(b)
        plsc.subcore_barrier()
        off = sid * rpt
        oout = pl.multiple_of(cid * npad + off, 8)
        for k in range(rpt // _C):
            pltpu.sync_copy(acc_sh.at[pl.ds(off + k * _C, _C)], rows0)
            pltpu.sync_copy(rows0, out_hbm.at[pl.ds(oout + k * _C, _C)])

    f = pl.kernel(
        body,
        out_type=jax.ShapeDtypeStruct((_NC * npad, d), dt),
        mesh=_sc_mesh(),
        compiler_params=pltpu.CompilerParams(use_tc_tiling_on_sc=False),
        scratch_types=[
            pltpu.VMEM((epw,), jnp.int32),
            pltpu.VMEM((epw,), jnp.int32),
            pltpu.VMEM((3, _C), jnp.int32),
            pltpu.VMEM((_C, d), dt),
            pltpu.VMEM((_C, d), dt),
            pltpu.VMEM((_C, d), dt),
            pltpu.VMEM((16, d), dt),
            pltpu.VMEM_SHARED((npad, d), dt),
            pltpu.SemaphoreType.DMA,
            pltpu.SemaphoreType.DMA,
            pltpu.SemaphoreType.DMA,
            pltpu.SemaphoreType.DMA,
            pltpu.SemaphoreType.DMA,
            pltpu.SemaphoreType.DMA,
        ],
    )
    return f(y, eidx)


def _tc1_body(x_ref, w_ref, d1_ref, y_ref):
    h = jnp.dot(x_ref[...], w_ref[...], preferred_element_type=jnp.float32)
    y_ref[...] = (h * d1_ref[...]).astype(y_ref.dtype)


def _combine_coarsen(acc_ref, y_ref, d_ref, b_ref):
    f32 = jnp.float32
    n, d = y_ref.shape
    npad = acc_ref.shape[0] // _NC
    s = (acc_ref[pl.ds(0, n), :].astype(f32)
         + acc_ref[pl.ds(npad, n), :].astype(f32)
         + y_ref[...].astype(f32))
    z = s * d_ref[...] + b_ref[...]
    z2 = z.reshape(n // 2, 2 * d)
    return jnp.maximum(jnp.maximum(z2[:, :d], z2[:, d:]), 0.0)


def _tc2_body(acc_ref, y1_ref, d1_ref, b1_ref, w2_ref, d2_ref, out_ref):
    h = _combine_coarsen(acc_ref, y1_ref, d1_ref, b1_ref)
    y2 = jnp.dot(h, w2_ref[...], preferred_element_type=jnp.float32)
    out_ref[...] = (y2 * d2_ref[...]).astype(out_ref.dtype)


def _tc3_body(acc_ref, y2_ref, d2_ref, b2_ref, seg_ref,
              fw1_ref, fb1_ref, fw2_ref, fb2_ref, out_ref):
    h = _combine_coarsen(acc_ref, y2_ref, d2_ref, b2_ref)
    oh = (lax.broadcasted_iota(jnp.int32, (8, h.shape[0]), 0)
          == seg_ref[...]).astype(jnp.float32)
    g = jnp.dot(oh, h, preferred_element_type=jnp.float32)
    g = jnp.maximum(
        jnp.dot(g, fw1_ref[...], preferred_element_type=jnp.float32)
        + fb1_ref[...], 0.0)
    out_ref[...] = jnp.maximum(
        jnp.dot(g, fw2_ref[...], preferred_element_type=jnp.float32)
        + fb2_ref[...], 0.0)


def kernel(x, edge_index, batch, W1, b1, W2, b2, fW1, fb1, fW2, fb2):
    n, d = x.shape
    f32 = jnp.float32
    bf16 = jnp.bfloat16

    deg2sc = _hist(edge_index, n)
    npadh = deg2sc.shape[0] // _NC
    deg_raw = (deg2sc[:npadh] + deg2sc[npadh:])[:n]
    dinv1 = lax.rsqrt(deg_raw + 1.0)
    degp = deg_raw.reshape(n // 2, 2)
    dinv2 = lax.rsqrt(degp[:, 0] + degp[:, 1] + 1.0)

    y1 = pl.pallas_call(
        _tc1_body, out_shape=jax.ShapeDtypeStruct((n, d), bf16),
    )(x, W1, dinv1[:, None])

    acc1 = _edge_scatter(y1, edge_index, shift=0, n_out=n)

    y2 = pl.pallas_call(
        _tc2_body, out_shape=jax.ShapeDtypeStruct((n // 2, d), bf16),
    )(acc1, y1, dinv1[:, None], b1.reshape(1, d), W2, dinv2[:, None])

    acc2 = _edge_scatter(y2, edge_index, shift=1, n_out=n // 2)

    seg = batch[::4].reshape(1, n // 4)
    out = pl.pallas_call(
        _tc3_body, out_shape=jax.ShapeDtypeStruct((8, d), f32),
    )(acc2, y2, dinv2[:, None], b2.reshape(1, d), seg,
      fW1, fb1.reshape(1, d), fW2, fb2.reshape(1, d))
    return out
